# baseline (speedup 1.0000x reference)
import functools

import jax
import jax.numpy as jnp
from jax import lax
from jax.experimental import pallas as pl
from jax.experimental.pallas import tpu as pltpu
from jax.experimental.pallas import tpu_sc as plsc

BATCH = 4096
SEQ = 50
DIM = 64

NUM_CORES = 2
NUM_SUBCORES = 16
NW = NUM_CORES * NUM_SUBCORES
BBLK = BATCH // NW
NBUF = 5

_mesh = plsc.VectorSubcoreMesh(core_axis_name="c", subcore_axis_name="s")


@functools.partial(
    pl.kernel,
    out_type=jax.ShapeDtypeStruct((BATCH, SEQ, DIM), jnp.float32),
    mesh=_mesh,
    compiler_params=pltpu.CompilerParams(use_tc_tiling_on_sc=False),
    scratch_types=[
        pltpu.VMEM((SEQ, BBLK), jnp.int32),
        [pltpu.VMEM((BBLK, DIM), jnp.float32) for _ in range(NBUF)],
        [pltpu.SemaphoreType.DMA for _ in range(NBUF)],
    ],
)
def _gather_rows(idx_hbm, table_hbm, out_hbm, idx_v, bufs, sems):
    wid = lax.axis_index("s") * NUM_CORES + lax.axis_index("c")
    wb = wid * BBLK
    pltpu.sync_copy(idx_hbm.at[:, pl.ds(wb, BBLK)], idx_v)

    def start_gather(s, b):
        pltpu.async_copy(table_hbm.at[idx_v.at[s]], bufs[b], sems[b])

    def finish(s, b):
        pltpu.make_async_copy(out_hbm.at[pl.ds(0, BBLK), 0], bufs[b], sems[b]).wait()
        pltpu.sync_copy(bufs[b], out_hbm.at[pl.ds(wb, BBLK), s])

    for b in range(NBUF):
        start_gather(b, b)

    @pl.loop(0, (SEQ - NBUF) // NBUF)
    def _main(g):
        for b in range(NBUF):
            s = g * NBUF + b
            finish(s, b)
            start_gather(s + NBUF, b)

    for b in range(NBUF):
        finish(SEQ - NBUF + b, b)


def kernel(IOs, table):
    return _gather_rows(IOs.T.astype(jnp.int32), table)
